# baseline (speedup 1.0000x reference)
import functools

import jax
import jax.numpy as jnp
from jax import lax
from jax.experimental import pallas as pl
from jax.experimental.pallas import tpu as pltpu
from jax.experimental.pallas import tpu_sc as plsc

NC = 2
NS = 16
NW = NC * NS
L = 16
K = 32
NB = 2


def _build(B, T, V, P, D):
    N = B * T
    per_w = N // NW
    G = per_w // K
    mesh = plsc.VectorSubcoreMesh(core_axis_name="c", subcore_axis_name="s")

    @functools.partial(
        pl.kernel,
        out_type=jax.ShapeDtypeStruct((N, D), jnp.float32),
        mesh=mesh,
        compiler_params=pltpu.CompilerParams(
            use_tc_tiling_on_sc=False, needs_layout_passes=False),
        scratch_types=[
            pltpu.VMEM((P, D), jnp.float32),
            pltpu.VMEM((per_w,), jnp.int32),
            pltpu.VMEM((per_w,), jnp.int32),
            pltpu.VMEM((K, D), jnp.float32),
            pltpu.VMEM((K, D), jnp.float32),
            pltpu.SemaphoreType.DMA((NB,)),
            pltpu.SemaphoreType.DMA((NB,)),
        ],
    )
    def sc_kernel(tok_hbm, pos_hbm, table_hbm, ptable_hbm, out_hbm,
                  ptab_v, tidx_v, pidx_v, b0, b1, gsem, ssem):
        bufs = [b0, b1]
        wid = lax.axis_index("s") * NC + lax.axis_index("c")
        base = wid * per_w
        pltpu.sync_copy(tok_hbm.at[wid], tidx_v)
        pltpu.sync_copy(pos_hbm.at[wid], pidx_v)
        pltpu.sync_copy(ptable_hbm, ptab_v)

        def start_gather(g, b):
            idx = tidx_v.at[pl.ds(g * K, K)]
            pltpu.async_copy(table_hbm.at[idx], bufs[b], gsem.at[b])

        def wait_gather(g, b):
            idx = tidx_v.at[pl.ds(g * K, K)]
            pltpu.make_async_copy(table_hbm.at[idx], bufs[b],
                                  gsem.at[b]).wait()

        def start_scatter(g, b):
            dst = out_hbm.at[pl.ds(base + g * K, K)]
            pltpu.async_copy(bufs[b], dst, ssem.at[b])

        def wait_scatter(g, b):
            dst = out_hbm.at[pl.ds(base + g * K, K)]
            pltpu.make_async_copy(bufs[b], dst, ssem.at[b]).wait()

        def compute(g, b):
            for rr in range(K // L):
                pids = pidx_v[pl.ds(g * K + rr * L, L)]
                for r in range(L):
                    pid = pids[r]
                    row = rr * L + r

                    @plsc.parallel_loop(0, D // L, unroll=8)
                    def cols(c):
                        v = ptab_v[pid, pl.ds(c * L, L)]
                        plsc.addupdate(bufs[b].at[row, pl.ds(c * L, L)], v)

        start_gather(0, 0)
        start_gather(1, 1)
        wait_gather(0, 0)
        compute(0, 0)
        start_scatter(0, 0)
        wait_scatter(0, 0)
        start_gather(2, 0)
        wait_gather(1, 1)
        compute(1, 1)
        start_scatter(1, 1)

        @pl.loop(0, (G - 3) // 2)
        def pair(i):
            for j in range(2):
                g = 2 + i * 2 + j
                b = j
                wait_scatter(g - 1, 1 - b)
                start_gather(g + 1, 1 - b)
                wait_gather(g, b)
                compute(g, b)
                start_scatter(g, b)

        wait_scatter(G - 2, 1)
        wait_gather(G - 1, 0)
        compute(G - 1, 0)
        start_scatter(G - 1, 0)
        wait_scatter(G - 1, 0)

    return sc_kernel


def kernel(input_ids, position_ids, token_table, position_table):
    B, T = input_ids.shape
    V, D = token_table.shape
    P = position_table.shape[0]
    N = B * T
    tok = input_ids.reshape(NW, N // NW).astype(jnp.int32)
    pos = position_ids.reshape(NW, N // NW).astype(jnp.int32)
    out = _build(B, T, V, P, D)(tok, pos, token_table, position_table)
    return out.reshape(B, T, D)


# speedup vs baseline: 1.0769x; 1.0769x over previous
import functools

import jax
import jax.numpy as jnp
from jax import lax
from jax.experimental import pallas as pl
from jax.experimental.pallas import tpu as pltpu
from jax.experimental.pallas import tpu_sc as plsc

NC = 2
NS = 16
NW = NC * NS
L = 16
K = 16
NB = 5
LA = 3


def _build(B, T, V, P, D):
    N = B * T
    per_w = N // NW
    G = per_w // K
    mesh = plsc.VectorSubcoreMesh(core_axis_name="c", subcore_axis_name="s")

    @functools.partial(
        pl.kernel,
        out_type=jax.ShapeDtypeStruct((N, D), jnp.float32),
        mesh=mesh,
        compiler_params=pltpu.CompilerParams(
            use_tc_tiling_on_sc=False, needs_layout_passes=False),
        scratch_types=[
            pltpu.VMEM((P, D), jnp.float32),
            pltpu.VMEM((per_w,), jnp.int32),
            pltpu.VMEM((per_w,), jnp.int32),
            pltpu.VMEM((K, D), jnp.float32),
            pltpu.VMEM((K, D), jnp.float32),
            pltpu.VMEM((K, D), jnp.float32),
            pltpu.VMEM((K, D), jnp.float32),
            pltpu.VMEM((K, D), jnp.float32),
            pltpu.SemaphoreType.DMA((NB,)),
            pltpu.SemaphoreType.DMA((NB,)),
        ],
    )
    def sc_kernel(tok_hbm, pos_hbm, table_hbm, ptable_hbm, out_hbm,
                  ptab_v, tidx_v, pidx_v, b0, b1, b2, b3, b4, gsem, ssem):
        bufs = [b0, b1, b2, b3, b4]
        wid = lax.axis_index("s") * NC + lax.axis_index("c")
        base = wid * per_w
        pltpu.sync_copy(tok_hbm.at[wid], tidx_v)
        pltpu.sync_copy(pos_hbm.at[wid], pidx_v)
        pltpu.sync_copy(ptable_hbm, ptab_v)

        def start_gather(g, b):
            idx = tidx_v.at[pl.ds(g * K, K)]
            pltpu.async_copy(table_hbm.at[idx], bufs[b], gsem.at[b])

        def wait_gather(g, b):
            idx = tidx_v.at[pl.ds(g * K, K)]
            pltpu.make_async_copy(table_hbm.at[idx], bufs[b],
                                  gsem.at[b]).wait()

        def start_scatter(g, b):
            dst = out_hbm.at[pl.ds(base + g * K, K)]
            pltpu.async_copy(bufs[b], dst, ssem.at[b])

        def wait_scatter(g, b):
            dst = out_hbm.at[pl.ds(base + g * K, K)]
            pltpu.make_async_copy(bufs[b], dst, ssem.at[b]).wait()

        def compute(g, b):
            pids = pidx_v[pl.ds(g * K, K)]
            for r in range(K):
                pid = pids[r]

                @plsc.parallel_loop(0, D // L, unroll=8)
                def cols(c):
                    v = ptab_v[pid, pl.ds(c * L, L)]
                    plsc.addupdate(bufs[b].at[r, pl.ds(c * L, L)], v)

        for p in range(LA):
            start_gather(p, p)
        for g in (0, 1):
            start_gather(g + LA, g + LA)
            wait_gather(g, g)
            compute(g, g)
            start_scatter(g, g)

        @pl.loop(0, (G - 4) // NB)
        def quint(i):
            for j in range(NB):
                g = 2 + i * NB + j
                b = (2 + j) % NB
                if j < NB - 1:
                    wait_scatter(g - (NB - LA), j)
                    start_gather(g + LA, j)
                else:
                    @pl.when(i < (G - 4) // NB - 1)
                    def _():
                        wait_scatter(g - (NB - LA), j)
                        start_gather(g + LA, j)
                wait_gather(g, b)
                compute(g, b)
                start_scatter(g, b)

        for g in (G - 2, G - 1):
            wait_gather(g, g % NB)
            compute(g, g % NB)
            start_scatter(g, g % NB)

        for g in range(G - NB, G):
            wait_scatter(g, g % NB)

    return sc_kernel


def kernel(input_ids, position_ids, token_table, position_table):
    B, T = input_ids.shape
    V, D = token_table.shape
    P = position_table.shape[0]
    N = B * T
    tok = input_ids.reshape(NW, N // NW).astype(jnp.int32)
    pos = position_ids.reshape(NW, N // NW).astype(jnp.int32)
    out = _build(B, T, V, P, D)(tok, pos, token_table, position_table)
    return out.reshape(B, T, D)
